# baseline (speedup 1.0000x reference)
import functools

import jax
import jax.numpy as jnp
from jax import lax
from jax.experimental import pallas as pl
from jax.experimental.pallas import tpu as pltpu
from jax.experimental.pallas import tpu_sc as plsc

EMB_ROWS = 100000
D = 768
BATCH = 4
SEQ = 2048
N_TOK = BATCH * SEQ
SCALE = float(D) ** 0.5

_info = plsc.get_sparse_core_info()
NC, NS, L = _info.num_cores, _info.num_subcores, _info.num_lanes
NW = NC * NS
S_PER_W = SEQ // NW
CH = 16
N_CHUNK = S_PER_W // CH
GROUPS_PER_ROW = D // L

_mesh = plsc.VectorSubcoreMesh(core_axis_name="c", subcore_axis_name="s")


@functools.partial(
    pl.kernel,
    mesh=_mesh,
    out_type=jax.ShapeDtypeStruct((N_TOK, D), jnp.float32),
    scratch_types=[
        pltpu.VMEM((BATCH * S_PER_W,), jnp.int32),
        pltpu.VMEM((BATCH, CH, D), jnp.float32),
        pltpu.VMEM((BATCH, CH, D), jnp.float32),
        pltpu.VMEM((CH, D), jnp.float32),
        pltpu.VMEM((CH, D), jnp.float32),
        pltpu.SemaphoreType.DMA,
        pltpu.SemaphoreType.DMA,
        pltpu.SemaphoreType.DMA,
        pltpu.SemaphoreType.DMA,
    ],
)
def _emb_lookup(xr_hbm, emb_hbm, pos_hbm, out_hbm,
                idx_v, buf0, buf1, pb0, pb1, g0, g1, w0, w1):
    wid = lax.axis_index("s") * NC + lax.axis_index("c")
    base = wid * S_PER_W
    bufs = (buf0, buf1)
    pbufs = (pb0, pb1)
    gsems = (g0, g1)
    wsems = (w0, w1)

    pltpu.sync_copy(xr_hbm.at[wid], idx_v)

    def _start_chunk(c, slot):
        cps = [pltpu.async_copy(
            pos_hbm.at[pl.ds(base + c * CH, CH), :], pbufs[slot], gsems[slot])]
        for b in range(BATCH):
            cps.append(pltpu.async_copy(
                emb_hbm.at[idx_v.at[pl.ds(b * S_PER_W + c * CH, CH)]],
                bufs[slot].at[b], gsems[slot]))
        return cps

    gathers = [None] * N_CHUNK
    writes = [None] * N_CHUNK
    gathers[0] = _start_chunk(0, 0)

    for c in range(N_CHUNK):
        cur = c % 2
        nxt = (c + 1) % 2
        for cp in gathers[c]:
            cp.wait()
        if c >= 1:
            for cp in writes[c - 1]:
                cp.wait()
        if c + 1 < N_CHUNK:
            gathers[c + 1] = _start_chunk(c + 1, nxt)

        buf = bufs[cur]
        pbuf = pbufs[cur]

        def _row_body(i, buf=buf, pbuf=pbuf):
            for j in range(GROUPS_PER_ROW):
                sl = pl.ds(j * L, L)
                pv = pbuf[i, sl]
                for b in range(BATCH):
                    buf[b, i, sl] = buf[b, i, sl] * SCALE + pv
        plsc.parallel_loop(0, CH, 1, unroll=2)(_row_body)

        writes[c] = [pltpu.async_copy(
            buf.at[b], out_hbm.at[pl.ds(b * SEQ + base + c * CH, CH), :],
            wsems[cur]) for b in range(BATCH)]

    for cp in writes[N_CHUNK - 1]:
        cp.wait()


def kernel(x, emb_weight, pos_weight):
    xr = (x.astype(jnp.int32)
           .reshape(BATCH, NW, S_PER_W)
           .swapaxes(0, 1)
           .reshape(NW, BATCH * S_PER_W))
    out = _emb_lookup(xr, emb_weight, pos_weight)
    return out.reshape(BATCH, SEQ, D)


# speedup vs baseline: 2.6456x; 2.6456x over previous
import functools

import jax
import jax.numpy as jnp
from jax import lax
from jax.experimental import pallas as pl
from jax.experimental.pallas import tpu as pltpu
from jax.experimental.pallas import tpu_sc as plsc

EMB_ROWS = 100000
D = 768
BATCH = 4
SEQ = 2048
N_TOK = BATCH * SEQ
SCALE = float(D) ** 0.5

_info = plsc.get_sparse_core_info()
NC, NS, L = _info.num_cores, _info.num_subcores, _info.num_lanes
NW = NC * NS
S_PER_W = SEQ // NW
CH = 32
N_CHUNK = BATCH * S_PER_W // CH
GROUPS_PER_ROW = D // L

_mesh = plsc.VectorSubcoreMesh(core_axis_name="c", subcore_axis_name="s")


@functools.partial(
    pl.kernel,
    mesh=_mesh,
    out_type=jax.ShapeDtypeStruct((N_TOK, D), jnp.float32),
    scratch_types=[
        pltpu.VMEM((BATCH * S_PER_W,), jnp.int32),
        pltpu.VMEM((S_PER_W, D), jnp.float32),
        pltpu.VMEM((CH, D), jnp.float32),
        pltpu.VMEM((CH, D), jnp.float32),
        pltpu.VMEM((CH, D), jnp.float32),
        pltpu.SemaphoreType.DMA,
        pltpu.SemaphoreType.DMA,
        pltpu.SemaphoreType.DMA,
        pltpu.SemaphoreType.DMA,
        pltpu.SemaphoreType.DMA,
        pltpu.SemaphoreType.DMA,
    ],
)
def _emb_lookup(xr_hbm, emb_hbm, pos_hbm, out_hbm,
                idx_v, pos_v, buf0, buf1, buf2, g0, g1, g2, w0, w1, w2):
    wid = lax.axis_index("s") * NC + lax.axis_index("c")
    base = wid * S_PER_W
    bufs = (buf0, buf1, buf2)
    gsems = (g0, g1, g2)
    wsems = (w0, w1, w2)

    pltpu.sync_copy(xr_hbm.at[wid], idx_v)

    def _idx(c):
        return idx_v.at[pl.ds(c * CH, CH)]

    gathers = [None] * N_CHUNK
    writes = [None] * N_CHUNK
    gathers[0] = pltpu.async_copy(emb_hbm.at[_idx(0)], bufs[0], gsems[0])

    pltpu.sync_copy(pos_hbm.at[pl.ds(base, S_PER_W), :], pos_v)

    for c in range(N_CHUNK):
        cur = c % 3
        nxt = (c + 1) % 3
        gathers[c].wait()
        if c + 1 < N_CHUNK:
            gathers[c + 1] = pltpu.async_copy(
                emb_hbm.at[_idx(c + 1)], bufs[nxt], gsems[nxt])

        b, h = divmod(c, 2)
        buf = bufs[cur]

        def _row_body(i, buf=buf, h=h):
            for j in range(GROUPS_PER_ROW):
                sl = pl.ds(j * L, L)
                buf[i, sl] = buf[i, sl] * SCALE + pos_v[h * CH + i, sl]
        plsc.parallel_loop(0, CH, 1, unroll=2)(_row_body)

        if c >= 1:
            writes[c - 1].wait()
        writes[c] = pltpu.async_copy(
            buf, out_hbm.at[pl.ds(b * SEQ + base + h * CH, CH), :], wsems[cur])

    writes[N_CHUNK - 1].wait()


def kernel(x, emb_weight, pos_weight):
    xr = (x.astype(jnp.int32)
           .reshape(BATCH, NW, S_PER_W)
           .swapaxes(0, 1)
           .reshape(NW, BATCH * S_PER_W))
    out = _emb_lookup(xr, emb_weight, pos_weight)
    return out.reshape(BATCH, SEQ, D)
